# baseline (speedup 1.0000x reference)
import functools
import math

import jax
import jax.numpy as jnp
import numpy as np
from jax import lax
from jax.experimental import pallas as pl
from jax.experimental.pallas import tpu as pltpu
from jax.experimental.pallas import tpu_sc as plsc

_NBINS = 10
_POOL = 8
_H = 512
_W = 512
_NIMG = 16
_HB = _H // _POOL
_WB = _W // _POOL
_TASK_ROWS = _POOL
_ACC = _NBINS * _WB
_NC = 2
_NS = 16
_NW = _NC * _NS
_CHUNK_TASKS = 8
_CHUNK_ROWS = _CHUNK_TASKS * _TASK_ROWS
_GROUP = 4


def _np_band():
    d = np.arange(_H)[:, None] - np.arange(_H)[None, :]
    vmat = np.where(d == 0, 2.0, np.where(np.abs(d) == 1, 1.0, 0.0))
    dmat = np.where(d == 1, 1.0, np.where(d == -1, -1.0, 0.0))
    return vmat.astype(np.float32), dmat.astype(np.float32)


_VMAT, _DMAT = _np_band()
_WBROW = (np.arange(_W, dtype=np.int32) // _POOL)[None, :] * 1


def _pack_body(x_ref, vm_ref, dm_ref, wb_ref, o_ref):
    img = x_ref[0].astype(jnp.bfloat16).astype(jnp.float32)

    t1 = lax.dot_general(vm_ref[...], img, (((1,), (0,)), ((), ())),
                         preferred_element_type=jnp.float32)
    t2 = lax.dot_general(dm_ref[...], img, (((1,), (0,)), ((), ())),
                         preferred_element_type=jnp.float32)

    zcol = jnp.zeros((_H, 1), dtype=jnp.float32)
    t1_l = jnp.concatenate([zcol, t1[:, :-1]], axis=1)
    t1_r = jnp.concatenate([t1[:, 1:], zcol], axis=1)
    t2_l = jnp.concatenate([zcol, t2[:, :-1]], axis=1)
    t2_r = jnp.concatenate([t2[:, 1:], zcol], axis=1)

    gx = t1_l - t1_r
    gy = t2_l + 2.0 * t2 + t2_r

    mag = jnp.sqrt(gx * gx + gy * gy) * jnp.float32(1.0 / (_POOL * _POOL))

    gxi = lax.bitcast_convert_type(gx, jnp.int32)
    gyi = lax.bitcast_convert_type(gy, jnp.int32)
    gyf = lax.bitcast_convert_type(
        jnp.bitwise_xor(gyi, jnp.bitwise_and(gxi, jnp.int32(-2147483648))),
        jnp.float32)
    v = jnp.where(gx == 0.0, jnp.abs(gy), gyf)
    ratio = v / jnp.abs(gx)

    bin_i = jnp.zeros((_H, _W), dtype=jnp.int32)
    for j in range(1, _NBINS):
        cj = ratio <= jnp.float32(1.0 / math.tan(j * math.pi / _NBINS))
        bin_i = bin_i + cj.astype(jnp.int32)

    loc = bin_i * _WB + wb_ref[...]
    magi = lax.bitcast_convert_type(mag, jnp.int32)
    o_ref[0] = jnp.bitwise_or(jnp.bitwise_and(magi, -1024), loc)


def _pack_stage(x2, img0):
    return pl.pallas_call(
        _pack_body,
        grid=(_GROUP,),
        in_specs=[pl.BlockSpec((1, _H, _W), lambda i: (i + img0, 0, 0)),
                  pl.BlockSpec((_H, _H), lambda i: (0, 0)),
                  pl.BlockSpec((_H, _H), lambda i: (0, 0)),
                  pl.BlockSpec((1, _W), lambda i: (0, 0))],
        out_specs=pl.BlockSpec((1, _H, _W), lambda i: (i, 0, 0)),
        out_shape=jax.ShapeDtypeStruct((_GROUP, _H, _W), jnp.int32),
    )(x2, jnp.asarray(_VMAT), jnp.asarray(_DMAT), jnp.asarray(_WBROW))


def _sc_body(packed_hbm, out_hbm, buf0, buf1, acc0, acc1,
             sin0, sin1, sout0, sout1):
    ntask = _GROUP * _HB
    tpw = ntask // _NW
    nchunk = max(tpw // _CHUNK_TASKS, 1)
    wpi = _HB // tpw
    wid = lax.axis_index("s") * _NC + lax.axis_index("c")
    img = wid // wpi
    hb0 = (wid - img * wpi) * tpw

    bufs = (buf0, buf1)
    sins = (sin0, sin1)
    accs = (acc0, acc1)
    souts = (sout0, sout1)

    def chunk_copy(cidx, slot):
        return pltpu.async_copy(
            packed_hbm.at[img, pl.ds(hb0 * _POOL + cidx * _CHUNK_ROWS,
                                     _CHUNK_ROWS)],
            bufs[slot], sins[slot])

    copies_in = [None, None]
    copies_in[0] = chunk_copy(0, 0)
    copies_out = [None, None]

    lane9 = lax.iota(jnp.int32, 16) * 9

    def scat(acc, v):
        l = jnp.bitwise_and(v, 1023)
        plsc.addupdate_scatter(acc, [l], plsc.bitcast(v, jnp.float32))

    def gather(buf, t8, j):
        row = t8 * _TASK_ROWS + lax.shift_right_logical(j, 5)
        col = jnp.bitwise_and(
            lane9 + lax.shift_left(jnp.bitwise_and(j, 31), 4), 511)
        rowv = jnp.broadcast_to(row, (16,))
        return plsc.load_gather(buf, [rowv, col])

    for cidx in range(nchunk):
        cslot = cidx % 2
        buf = bufs[cslot]
        copies_in[cslot].wait()
        if cidx + 1 < nchunk:
            copies_in[(cidx + 1) % 2] = chunk_copy(cidx + 1, (cidx + 1) % 2)

        for t8 in range(_CHUNK_TASKS):
            tt = cidx * _CHUNK_TASKS + t8
            aslot = tt % 2
            acc = accs[aslot]
            if copies_out[aslot] is not None:
                for h in copies_out[aslot]:
                    h.wait()

            def zero(i, _c, acc=acc):
                acc[pl.ds(i * 16, 16)] = jnp.zeros((16,), jnp.float32)
                return _c
            lax.fori_loop(0, _ACC // 16, zero, 0, unroll=4)

            def inner(j, carry, buf=buf, acc=acc, t8=t8):
                va, vb = carry
                vc = gather(buf, t8, j + 2)
                scat(acc, va)
                return (vb, vc)

            va, vb = lax.fori_loop(
                0, _TASK_ROWS * 32 - 2, inner,
                (gather(buf, t8, 0), gather(buf, t8, 1)), unroll=4)
            scat(acc, va)
            scat(acc, vb)

            hb = hb0 + tt
            obase = (img * _NBINS * _HB + hb) * _WB
            outs = []
            for b in range(_NBINS):
                outs.append(pltpu.async_copy(
                    acc.at[pl.ds(b * _WB, _WB)],
                    out_hbm.at[pl.ds(obase + b * _HB * _WB, _WB)],
                    souts[aslot]))
            copies_out[aslot] = outs

    for slot in range(2):
        if copies_out[slot] is not None:
            for h in copies_out[slot]:
                h.wait()


@functools.cache
def _sc_hist():
    return pl.kernel(
        _sc_body,
        out_type=jax.ShapeDtypeStruct((_GROUP * _NBINS * _HB * _WB,),
                                      jnp.float32),
        mesh=plsc.VectorSubcoreMesh(core_axis_name="c", subcore_axis_name="s"),
        scratch_types=[
            pltpu.VMEM((_CHUNK_ROWS, _W), jnp.int32),
            pltpu.VMEM((_CHUNK_ROWS, _W), jnp.int32),
            pltpu.VMEM((_ACC,), jnp.float32),
            pltpu.VMEM((_ACC,), jnp.float32),
            pltpu.SemaphoreType.DMA,
            pltpu.SemaphoreType.DMA,
            pltpu.SemaphoreType.DMA,
            pltpu.SemaphoreType.DMA,
        ],
        compiler_params=pltpu.CompilerParams(needs_layout_passes=False),
    )


@jax.jit
def kernel(x):
    x2 = x.reshape(_NIMG, _H, _W)
    outs = []
    for g0 in range(0, _NIMG, _GROUP):
        packed = _pack_stage(x2, g0)
        hist = _sc_hist()(packed)
        outs.append(hist.reshape(_GROUP, _NBINS, _HB, _WB))
    return jnp.concatenate(outs, axis=0)


# speedup vs baseline: 1.0163x; 1.0163x over previous
import functools
import math

import jax
import jax.numpy as jnp
from jax import lax
from jax.experimental import pallas as pl
from jax.experimental.pallas import tpu as pltpu
from jax.experimental.pallas import tpu_sc as plsc

_NBINS = 10
_POOL = 8
_H = 512
_W = 512
_NIMG = 16
_HB = _H // _POOL
_WB = _W // _POOL
_TASK_ROWS = _POOL
_ACC = _NBINS * _WB
_NC = 2
_NS = 16
_NW = _NC * _NS
_CHUNK_TASKS = 8
_CHUNK_ROWS = _CHUNK_TASKS * _TASK_ROWS
_GROUP = 4


def _pack_body(x_ref, o_ref):
    img = x_ref[0].astype(jnp.bfloat16).astype(jnp.float32)

    r = lax.broadcasted_iota(jnp.int32, (_H, _H), 0)
    c = lax.broadcasted_iota(jnp.int32, (_H, _H), 1)
    d = r - c
    vmatv = jnp.where(d == 0, 2.0, jnp.where(jnp.abs(d) == 1, 1.0, 0.0))
    dmat = jnp.where(d == 1, 1.0, jnp.where(d == -1, -1.0, 0.0))
    t1 = lax.dot_general(vmatv, img, (((1,), (0,)), ((), ())),
                         preferred_element_type=jnp.float32)
    t2 = lax.dot_general(dmat, img, (((1,), (0,)), ((), ())),
                         preferred_element_type=jnp.float32)

    zcol = jnp.zeros((_H, 1), dtype=jnp.float32)
    t1_l = jnp.concatenate([zcol, t1[:, :-1]], axis=1)
    t1_r = jnp.concatenate([t1[:, 1:], zcol], axis=1)
    t2_l = jnp.concatenate([zcol, t2[:, :-1]], axis=1)
    t2_r = jnp.concatenate([t2[:, 1:], zcol], axis=1)

    gx = t1_l - t1_r
    gy = t2_l + 2.0 * t2 + t2_r

    mag = jnp.sqrt(gx * gx + gy * gy) * jnp.float32(1.0 / (_POOL * _POOL))

    s = jnp.where(gx > 0.0, 1.0, jnp.where(gx < 0.0, -1.0,
                  jnp.where(gy < 0.0, -1.0, 1.0))).astype(jnp.float32)
    ratio = (s * gy) / jnp.abs(gx)

    bin_i = jnp.zeros((_H, _W), dtype=jnp.int32)
    for j in range(1, _NBINS):
        cj = ratio <= jnp.float32(1.0 / math.tan(j * math.pi / _NBINS))
        bin_i = bin_i + cj.astype(jnp.int32)

    wb = lax.broadcasted_iota(jnp.int32, (_H, _W), 1) // _POOL
    loc = bin_i * _WB + wb
    magi = lax.bitcast_convert_type(mag, jnp.int32)
    o_ref[0] = jnp.bitwise_or(jnp.bitwise_and(magi, -1024), loc)


def _pack_stage(x2, img0):
    return pl.pallas_call(
        _pack_body,
        grid=(_GROUP,),
        in_specs=[pl.BlockSpec((1, _H, _W), lambda i: (i + img0, 0, 0))],
        out_specs=pl.BlockSpec((1, _H, _W), lambda i: (i, 0, 0)),
        out_shape=jax.ShapeDtypeStruct((_GROUP, _H, _W), jnp.int32),
    )(x2)


def _sc_body(packed_hbm, out_hbm, buf0, buf1, acc0, acc1,
             sin0, sin1, sout0, sout1):
    ntask = _GROUP * _HB
    tpw = ntask // _NW
    nchunk = max(tpw // _CHUNK_TASKS, 1)
    wpi = _HB // tpw
    wid = lax.axis_index("s") * _NC + lax.axis_index("c")
    img = wid // wpi
    hb0 = (wid - img * wpi) * tpw

    bufs = (buf0, buf1)
    sins = (sin0, sin1)
    accs = (acc0, acc1)
    souts = (sout0, sout1)

    def chunk_copy(cidx, slot):
        return pltpu.async_copy(
            packed_hbm.at[img, pl.ds(hb0 * _POOL + cidx * _CHUNK_ROWS,
                                     _CHUNK_ROWS)],
            bufs[slot], sins[slot])

    copies_in = [None, None]
    copies_in[0] = chunk_copy(0, 0)
    copies_out = [None, None]

    lane9 = lax.iota(jnp.int32, 16) * 9

    def scat(acc, v):
        l = jnp.bitwise_and(v, 1023)
        plsc.addupdate_scatter(acc, [l], plsc.bitcast(v, jnp.float32))

    def gather(buf, t8, j):
        row = t8 * _TASK_ROWS + lax.shift_right_logical(j, 5)
        col = jnp.bitwise_and(
            lane9 + lax.shift_left(jnp.bitwise_and(j, 31), 4), 511)
        rowv = jnp.broadcast_to(row, (16,))
        return plsc.load_gather(buf, [rowv, col])

    for cidx in range(nchunk):
        cslot = cidx % 2
        buf = bufs[cslot]
        copies_in[cslot].wait()
        if cidx + 1 < nchunk:
            copies_in[(cidx + 1) % 2] = chunk_copy(cidx + 1, (cidx + 1) % 2)

        for t8 in range(_CHUNK_TASKS):
            tt = cidx * _CHUNK_TASKS + t8
            aslot = tt % 2
            acc = accs[aslot]
            if copies_out[aslot] is not None:
                for h in copies_out[aslot]:
                    h.wait()

            def zero(i, _c, acc=acc):
                acc[pl.ds(i * 16, 16)] = jnp.zeros((16,), jnp.float32)
                return _c
            lax.fori_loop(0, _ACC // 16, zero, 0, unroll=4)

            def inner(j, carry, buf=buf, acc=acc, t8=t8):
                va, vb = carry
                vc = gather(buf, t8, j + 2)
                scat(acc, va)
                return (vb, vc)

            va, vb = lax.fori_loop(
                0, _TASK_ROWS * 32 - 2, inner,
                (gather(buf, t8, 0), gather(buf, t8, 1)), unroll=4)
            scat(acc, va)
            scat(acc, vb)

            hb = hb0 + tt
            obase = (img * _NBINS * _HB + hb) * _WB
            outs = []
            for b in range(_NBINS):
                outs.append(pltpu.async_copy(
                    acc.at[pl.ds(b * _WB, _WB)],
                    out_hbm.at[pl.ds(obase + b * _HB * _WB, _WB)],
                    souts[aslot]))
            copies_out[aslot] = outs

    for slot in range(2):
        if copies_out[slot] is not None:
            for h in copies_out[slot]:
                h.wait()


@functools.cache
def _sc_hist():
    return pl.kernel(
        _sc_body,
        out_type=jax.ShapeDtypeStruct((_GROUP * _NBINS * _HB * _WB,),
                                      jnp.float32),
        mesh=plsc.VectorSubcoreMesh(core_axis_name="c", subcore_axis_name="s"),
        scratch_types=[
            pltpu.VMEM((_CHUNK_ROWS, _W), jnp.int32),
            pltpu.VMEM((_CHUNK_ROWS, _W), jnp.int32),
            pltpu.VMEM((_ACC,), jnp.float32),
            pltpu.VMEM((_ACC,), jnp.float32),
            pltpu.SemaphoreType.DMA,
            pltpu.SemaphoreType.DMA,
            pltpu.SemaphoreType.DMA,
            pltpu.SemaphoreType.DMA,
        ],
        compiler_params=pltpu.CompilerParams(needs_layout_passes=False),
    )


@jax.jit
def kernel(x):
    x2 = x.reshape(_NIMG, _H, _W)
    outs = []
    for g0 in range(0, _NIMG, _GROUP):
        packed = _pack_stage(x2, g0)
        hist = _sc_hist()(packed)
        outs.append(hist.reshape(_GROUP, _NBINS, _HB, _WB))
    return jnp.concatenate(outs, axis=0)
